# baseline (speedup 1.0000x reference)
import functools

import jax
import jax.numpy as jnp
from jax import lax
from jax.experimental import pallas as pl
from jax.experimental.pallas import tpu as pltpu
from jax.experimental.pallas import tpu_sc as plsc

N = 10000
NP = 10240
D = 256
HALF = 128
E = 160000
NC = 2
NS = 16
L = 16
PAD_NODE = N
CH = 80
EP = NS * CH * 128
EPW = EP // (NC * NS)
RPT = NP // NS

_SC_MESH = plsc.VectorSubcoreMesh(core_axis_name="c", subcore_axis_name="s")


def _deg_body(dst_hbm, out_hbm, hist_v, idx_v):
    c = lax.axis_index("c")
    s = lax.axis_index("s")
    w = c * NS + s

    def _zero(i, _):
        hist_v[pl.ds(i * L, L)] = jnp.zeros((L,), jnp.float32)
        return 0

    lax.fori_loop(0, NP // L, _zero, 0)

    pltpu.sync_copy(dst_hbm.at[pl.ds(w * EPW, EPW)], idx_v)

    ones = jnp.ones((L,), jnp.float32)

    def _hist(i, _):
        idx = idx_v[pl.ds(i * L, L)]
        plsc.addupdate_scatter(hist_v, [idx], ones)
        return 0

    lax.fori_loop(0, EPW // L, _hist, 0)
    pltpu.sync_copy(hist_v, out_hbm.at[w])


_deg_kernel = functools.partial(
    pl.kernel,
    out_type=jax.ShapeDtypeStruct((NC * NS, NP), jnp.float32),
    mesh=_SC_MESH,
    compiler_params=pltpu.CompilerParams(needs_layout_passes=False),
    scratch_types=[
        pltpu.VMEM((NP,), jnp.float32),
        pltpu.VMEM((EPW,), jnp.int32),
    ],
)(_deg_body)


def _copy_idx_row(src_ref, dst_ref):
    for q in range(128 // L):
        dst_ref[0, pl.ds(q * L, L)] = src_ref[1, pl.ds(q * L, L)]


def _scatter_body(hp_hbm, ei_hbm, out_hbm, acc_sh, rows_a, rows_b, ia, ib,
                  sca, scb, gs_a, gs_b, ss_a, ss_b, is_a, is_b):
    c = lax.axis_index("c")
    s = lax.axis_index("s")
    hp2d = hp_hbm.at[c]

    pltpu.sync_copy(hp2d.at[pl.ds(s * RPT, RPT)], acc_sh.at[pl.ds(s * RPT, RPT)])
    plsc.subcore_barrier()

    pltpu.sync_copy(ei_hbm.at[s, 0], ia)
    pltpu.async_copy(hp2d.at[ia.at[0]], rows_a, gs_a)
    pltpu.sync_copy(ei_hbm.at[s, 1], ib)
    pltpu.async_copy(hp2d.at[ib.at[0]], rows_b, gs_b)

    def _pair(t, _):
        jj = 2 * t
        pltpu.make_async_copy(hp2d.at[ia.at[0]], rows_a, gs_a).wait()
        _copy_idx_row(ia, sca)
        pltpu.async_copy(rows_a, acc_sh.at[sca.at[0]], ss_a, add=True)

        @pl.when(jj + 2 < CH)
        def _():
            pltpu.async_copy(ei_hbm.at[s, jj + 2], ia, is_a)

        pltpu.make_async_copy(hp2d.at[ib.at[0]], rows_b, gs_b).wait()
        _copy_idx_row(ib, scb)
        pltpu.async_copy(rows_b, acc_sh.at[scb.at[0]], ss_b, add=True)

        @pl.when(jj + 3 < CH)
        def _():
            pltpu.async_copy(ei_hbm.at[s, jj + 3], ib, is_b)

        @pl.when(jj + 2 < CH)
        def _():
            pltpu.make_async_copy(rows_a, acc_sh.at[sca.at[0]], ss_a).wait()
            pltpu.make_async_copy(ei_hbm.at[s, 0], ia, is_a).wait()
            pltpu.async_copy(hp2d.at[ia.at[0]], rows_a, gs_a)

        @pl.when(jj + 3 < CH)
        def _():
            pltpu.make_async_copy(rows_b, acc_sh.at[scb.at[0]], ss_b).wait()
            pltpu.make_async_copy(ei_hbm.at[s, 0], ib, is_b).wait()
            pltpu.async_copy(hp2d.at[ib.at[0]], rows_b, gs_b)

        return 0

    lax.fori_loop(0, CH // 2, _pair, 0)
    pltpu.make_async_copy(rows_a, acc_sh.at[sca.at[0]], ss_a).wait()
    pltpu.make_async_copy(rows_b, acc_sh.at[scb.at[0]], ss_b).wait()
    plsc.subcore_barrier()
    pltpu.sync_copy(acc_sh.at[pl.ds(s * RPT, RPT)], out_hbm.at[c, pl.ds(s * RPT, RPT)])


_scatter_kernel = functools.partial(
    pl.kernel,
    out_type=jax.ShapeDtypeStruct((NC, NP, HALF), jnp.float32),
    mesh=_SC_MESH,
    compiler_params=pltpu.CompilerParams(needs_layout_passes=False),
    scratch_types=[
        pltpu.VMEM_SHARED((NP, HALF), jnp.float32),
        pltpu.VMEM((128, HALF), jnp.float32),
        pltpu.VMEM((128, HALF), jnp.float32),
        pltpu.VMEM((2, 128), jnp.int32),
        pltpu.VMEM((2, 128), jnp.int32),
        pltpu.VMEM((1, 128), jnp.int32),
        pltpu.VMEM((1, 128), jnp.int32),
        pltpu.SemaphoreType.DMA,
        pltpu.SemaphoreType.DMA,
        pltpu.SemaphoreType.DMA,
        pltpu.SemaphoreType.DMA,
        pltpu.SemaphoreType.DMA,
        pltpu.SemaphoreType.DMA,
    ],
)(_scatter_body)


_BN = 1024
_GRID = NP // _BN


def _dinv_of(degp_ref):
    return lax.rsqrt(jnp.sum(degp_ref[...], axis=0) + 1.0)


def _mm1_body(degp_ref, x_ref, w_ref, out_ref):
    dinv = _dinv_of(degp_ref)
    h = jnp.dot(x_ref[...], w_ref[...], preferred_element_type=jnp.float32)
    h = h * dinv[:, None]
    out_ref[0] = h[:, :HALF]
    out_ref[1] = h[:, HALF:]


def _tc_mm1(degp, x_pad, w1):
    return pl.pallas_call(
        _mm1_body,
        grid=(_GRID,),
        in_specs=[
            pl.BlockSpec((NC * NS, _BN), lambda i: (0, i)),
            pl.BlockSpec((_BN, D), lambda i: (i, 0)),
            pl.BlockSpec((D, D), lambda i: (0, 0)),
        ],
        out_specs=pl.BlockSpec((NC, _BN, HALF), lambda i: (0, i, 0)),
        out_shape=jax.ShapeDtypeStruct((NC, NP, HALF), jnp.float32),
    )(degp, x_pad, w1)


def _mid_body(degp_ref, agg_ref, b_ref, w_ref, out_ref):
    dinv = _dinv_of(degp_ref)
    agg = jnp.concatenate([agg_ref[0], agg_ref[1]], axis=-1)
    h1 = jax.nn.relu(agg * dinv[:, None] + b_ref[0, :])
    h2 = jnp.dot(h1, w_ref[...], preferred_element_type=jnp.float32)
    h2 = h2 * dinv[:, None]
    out_ref[0] = h2[:, :HALF]
    out_ref[1] = h2[:, HALF:]


def _tc_mid(degp, agg, b1, w2):
    return pl.pallas_call(
        _mid_body,
        grid=(_GRID,),
        in_specs=[
            pl.BlockSpec((NC * NS, _BN), lambda i: (0, i)),
            pl.BlockSpec((NC, _BN, HALF), lambda i: (0, i, 0)),
            pl.BlockSpec((1, D), lambda i: (0, 0)),
            pl.BlockSpec((D, D), lambda i: (0, 0)),
        ],
        out_specs=pl.BlockSpec((NC, _BN, HALF), lambda i: (0, i, 0)),
        out_shape=jax.ShapeDtypeStruct((NC, NP, HALF), jnp.float32),
    )(degp, agg, b1, w2)


def _head_body(degp_ref, agg_ref, b_ref, wf1_ref, bf1_ref, wf2_ref, bf2_ref,
               out_ref):
    dinv = _dinv_of(degp_ref)
    agg = jnp.concatenate([agg_ref[0], agg_ref[1]], axis=-1)
    h2 = jax.nn.relu(agg * dinv[:, None] + b_ref[0, :])
    f1 = jax.nn.relu(
        jnp.dot(h2, wf1_ref[...], preferred_element_type=jnp.float32)
        + bf1_ref[0, :])
    out_ref[...] = (
        jnp.dot(f1, wf2_ref[...], preferred_element_type=jnp.float32)
        + bf2_ref[0, :])


def _tc_head(degp, agg, b2, wf1, bf1, wf2, bf2):
    return pl.pallas_call(
        _head_body,
        grid=(_GRID,),
        in_specs=[
            pl.BlockSpec((NC * NS, _BN), lambda i: (0, i)),
            pl.BlockSpec((NC, _BN, HALF), lambda i: (0, i, 0)),
            pl.BlockSpec((1, D), lambda i: (0, 0)),
            pl.BlockSpec((D, HALF), lambda i: (0, 0)),
            pl.BlockSpec((1, HALF), lambda i: (0, 0)),
            pl.BlockSpec((HALF, 64), lambda i: (0, 0)),
            pl.BlockSpec((1, 64), lambda i: (0, 0)),
        ],
        out_specs=pl.BlockSpec((_BN, 64), lambda i: (i, 0)),
        out_shape=jax.ShapeDtypeStruct((NP, 64), jnp.float32),
    )(degp, agg, b2, wf1, bf1, wf2, bf2)


def kernel(x, edge_index, W1, b1, W2, b2, Wf1, bf1, Wf2, bf2):
    src = edge_index[0]
    dst = edge_index[1]
    pad = jnp.full((EP - E,), PAD_NODE, jnp.int32)
    src_flat = jnp.concatenate([src, pad])
    dst_flat = jnp.concatenate([dst, pad])
    src3 = src_flat.reshape(NS, CH, 128)
    dst3 = dst_flat.reshape(NS, CH, 128)
    ei3 = jnp.stack([src3, dst3], axis=2)
    x_pad = jnp.pad(x, ((0, NP - N), (0, 0)))

    degp = _deg_kernel(dst_flat)

    hp1 = _tc_mm1(degp, x_pad, W1)
    agg1 = _scatter_kernel(hp1, ei3)
    hp2 = _tc_mid(degp, agg1, b1.reshape(1, D), W2)
    agg2 = _scatter_kernel(hp2, ei3)
    out = _tc_head(degp, agg2, b2.reshape(1, D), Wf1.astype(jnp.float32),
                   bf1.reshape(1, HALF), Wf2, bf2.reshape(1, 64))
    return out[:N]


# speedup vs baseline: 1.7311x; 1.7311x over previous
import functools

import jax
import jax.numpy as jnp
from jax import lax
from jax.experimental import pallas as pl
from jax.experimental.pallas import tpu as pltpu
from jax.experimental.pallas import tpu_sc as plsc

N = 10000
NP = 10240
D = 256
QT = 64
E = 160000
NC = 2
NS = 16
L = 16
PAD_NODE = N
CR = 128
CH = 80
EP = NS * CH * CR
EPW = EP // (NC * NS)
RPT = NP // NS

_SC_MESH = plsc.VectorSubcoreMesh(core_axis_name="c", subcore_axis_name="s")


def _deg_body(dst_hbm, out_hbm, hist_v, idx_v):
    c = lax.axis_index("c")
    s = lax.axis_index("s")
    w = c * NS + s

    def _zero(i, _):
        hist_v[pl.ds(i * L, L)] = jnp.zeros((L,), jnp.float32)
        return 0

    lax.fori_loop(0, NP // L, _zero, 0)

    pltpu.sync_copy(dst_hbm.at[pl.ds(w * EPW, EPW)], idx_v)

    ones = jnp.ones((L,), jnp.float32)

    def _hist(i, _):
        idx = idx_v[pl.ds(i * L, L)]
        plsc.addupdate_scatter(hist_v, [idx], ones)
        return 0

    lax.fori_loop(0, EPW // L, _hist, 0)
    pltpu.sync_copy(hist_v, out_hbm.at[w])


_deg_kernel = functools.partial(
    pl.kernel,
    out_type=jax.ShapeDtypeStruct((NC * NS, NP), jnp.float32),
    mesh=_SC_MESH,
    compiler_params=pltpu.CompilerParams(needs_layout_passes=False),
    scratch_types=[
        pltpu.VMEM((NP,), jnp.float32),
        pltpu.VMEM((EPW,), jnp.int32),
    ],
)(_deg_body)


def _copy_idx_row(src_ref, dst_ref):
    for q in range(CR // L):
        dst_ref[0, pl.ds(q * L, L)] = src_ref[1, pl.ds(q * L, L)]


def _scatter_body(hp_hbm, ei_hbm, out_hbm, table_sh, acc_sh, rows_a, rows_b,
                  ia, ib, sca, scb, gs_a, gs_b, ss_a, ss_b, is_a, is_b):
    c = lax.axis_index("c")
    s = lax.axis_index("s")
    rsl = pl.ds(s * RPT, RPT)

    for p in range(2):
        hp3 = hp_hbm.at[c, p]
        pltpu.sync_copy(hp3.at[rsl], table_sh.at[rsl])
        pltpu.sync_copy(hp3.at[rsl], acc_sh.at[rsl])
        plsc.subcore_barrier()

        pltpu.sync_copy(ei_hbm.at[s, 0], ia)
        pltpu.async_copy(table_sh.at[ia.at[0]], rows_a, gs_a)
        pltpu.sync_copy(ei_hbm.at[s, 1], ib)
        pltpu.async_copy(table_sh.at[ib.at[0]], rows_b, gs_b)

        def _pair(t, _):
            jj = 2 * t
            pltpu.make_async_copy(table_sh.at[ia.at[0]], rows_a, gs_a).wait()
            _copy_idx_row(ia, sca)
            pltpu.async_copy(rows_a, acc_sh.at[sca.at[0]], ss_a, add=True)

            @pl.when(jj + 2 < CH)
            def _():
                pltpu.async_copy(ei_hbm.at[s, jj + 2], ia, is_a)

            pltpu.make_async_copy(table_sh.at[ib.at[0]], rows_b, gs_b).wait()
            _copy_idx_row(ib, scb)
            pltpu.async_copy(rows_b, acc_sh.at[scb.at[0]], ss_b, add=True)

            @pl.when(jj + 3 < CH)
            def _():
                pltpu.async_copy(ei_hbm.at[s, jj + 3], ib, is_b)

            @pl.when(jj + 2 < CH)
            def _():
                pltpu.make_async_copy(rows_a, acc_sh.at[sca.at[0]], ss_a).wait()
                pltpu.make_async_copy(ei_hbm.at[s, 0], ia, is_a).wait()
                pltpu.async_copy(table_sh.at[ia.at[0]], rows_a, gs_a)

            @pl.when(jj + 3 < CH)
            def _():
                pltpu.make_async_copy(rows_b, acc_sh.at[scb.at[0]], ss_b).wait()
                pltpu.make_async_copy(ei_hbm.at[s, 0], ib, is_b).wait()
                pltpu.async_copy(table_sh.at[ib.at[0]], rows_b, gs_b)

            return 0

        lax.fori_loop(0, CH // 2, _pair, 0)
        pltpu.make_async_copy(rows_a, acc_sh.at[sca.at[0]], ss_a).wait()
        pltpu.make_async_copy(rows_b, acc_sh.at[scb.at[0]], ss_b).wait()
        plsc.subcore_barrier()
        pltpu.sync_copy(acc_sh.at[rsl], out_hbm.at[c, p, rsl])


_scatter_kernel = functools.partial(
    pl.kernel,
    out_type=jax.ShapeDtypeStruct((NC, 2, NP, QT), jnp.float32),
    mesh=_SC_MESH,
    compiler_params=pltpu.CompilerParams(needs_layout_passes=False),
    scratch_types=[
        pltpu.VMEM_SHARED((NP, QT), jnp.float32),
        pltpu.VMEM_SHARED((NP, QT), jnp.float32),
        pltpu.VMEM((CR, QT), jnp.float32),
        pltpu.VMEM((CR, QT), jnp.float32),
        pltpu.VMEM((2, CR), jnp.int32),
        pltpu.VMEM((2, CR), jnp.int32),
        pltpu.VMEM((1, CR), jnp.int32),
        pltpu.VMEM((1, CR), jnp.int32),
        pltpu.SemaphoreType.DMA,
        pltpu.SemaphoreType.DMA,
        pltpu.SemaphoreType.DMA,
        pltpu.SemaphoreType.DMA,
        pltpu.SemaphoreType.DMA,
        pltpu.SemaphoreType.DMA,
    ],
)(_scatter_body)


_BN = 1024
_GRID = NP // _BN


def _dinv_of(degp_ref):
    return lax.rsqrt(jnp.sum(degp_ref[...], axis=0) + 1.0)


def _split4(h, out_ref):
    for c in range(NC):
        for p in range(2):
            q = c * 2 + p
            out_ref[c, p] = h[:, q * QT:(q + 1) * QT]


def _join4(agg_ref):
    return jnp.concatenate(
        [agg_ref[c, p] for c in range(NC) for p in range(2)], axis=-1)


def _mm1_body(degp_ref, x_ref, w_ref, out_ref):
    dinv = _dinv_of(degp_ref)
    h = jnp.dot(x_ref[...], w_ref[...], preferred_element_type=jnp.float32)
    _split4(h * dinv[:, None], out_ref)


def _tc_mm1(degp, x_pad, w1):
    return pl.pallas_call(
        _mm1_body,
        grid=(_GRID,),
        in_specs=[
            pl.BlockSpec((NC * NS, _BN), lambda i: (0, i)),
            pl.BlockSpec((_BN, D), lambda i: (i, 0)),
            pl.BlockSpec((D, D), lambda i: (0, 0)),
        ],
        out_specs=pl.BlockSpec((NC, 2, _BN, QT), lambda i: (0, 0, i, 0)),
        out_shape=jax.ShapeDtypeStruct((NC, 2, NP, QT), jnp.float32),
    )(degp, x_pad, w1)


def _mid_body(degp_ref, agg_ref, b_ref, w_ref, out_ref):
    dinv = _dinv_of(degp_ref)
    agg = _join4(agg_ref)
    h1 = jax.nn.relu(agg * dinv[:, None] + b_ref[0, :])
    h2 = jnp.dot(h1, w_ref[...], preferred_element_type=jnp.float32)
    _split4(h2 * dinv[:, None], out_ref)


def _tc_mid(degp, agg, b1, w2):
    return pl.pallas_call(
        _mid_body,
        grid=(_GRID,),
        in_specs=[
            pl.BlockSpec((NC * NS, _BN), lambda i: (0, i)),
            pl.BlockSpec((NC, 2, _BN, QT), lambda i: (0, 0, i, 0)),
            pl.BlockSpec((1, D), lambda i: (0, 0)),
            pl.BlockSpec((D, D), lambda i: (0, 0)),
        ],
        out_specs=pl.BlockSpec((NC, 2, _BN, QT), lambda i: (0, 0, i, 0)),
        out_shape=jax.ShapeDtypeStruct((NC, 2, NP, QT), jnp.float32),
    )(degp, agg, b1, w2)


def _head_body(degp_ref, agg_ref, b_ref, wf1_ref, bf1_ref, wf2_ref, bf2_ref,
               out_ref):
    dinv = _dinv_of(degp_ref)
    agg = _join4(agg_ref)
    h2 = jax.nn.relu(agg * dinv[:, None] + b_ref[0, :])
    f1 = jax.nn.relu(
        jnp.dot(h2, wf1_ref[...], preferred_element_type=jnp.float32)
        + bf1_ref[0, :])
    out_ref[...] = (
        jnp.dot(f1, wf2_ref[...], preferred_element_type=jnp.float32)
        + bf2_ref[0, :])


def _tc_head(degp, agg, b2, wf1, bf1, wf2, bf2):
    return pl.pallas_call(
        _head_body,
        grid=(_GRID,),
        in_specs=[
            pl.BlockSpec((NC * NS, _BN), lambda i: (0, i)),
            pl.BlockSpec((NC, 2, _BN, QT), lambda i: (0, 0, i, 0)),
            pl.BlockSpec((1, D), lambda i: (0, 0)),
            pl.BlockSpec((D, 128), lambda i: (0, 0)),
            pl.BlockSpec((1, 128), lambda i: (0, 0)),
            pl.BlockSpec((128, 64), lambda i: (0, 0)),
            pl.BlockSpec((1, 64), lambda i: (0, 0)),
        ],
        out_specs=pl.BlockSpec((_BN, 64), lambda i: (i, 0)),
        out_shape=jax.ShapeDtypeStruct((NP, 64), jnp.float32),
    )(degp, agg, b2, wf1, bf1, wf2, bf2)


def kernel(x, edge_index, W1, b1, W2, b2, Wf1, bf1, Wf2, bf2):
    src = edge_index[0]
    dst = edge_index[1]
    pad = jnp.full((EP - E,), PAD_NODE, jnp.int32)
    src_flat = jnp.concatenate([src, pad])
    dst_flat = jnp.concatenate([dst, pad])
    src3 = src_flat.reshape(NS, CH, CR)
    dst3 = dst_flat.reshape(NS, CH, CR)
    ei3 = jnp.stack([src3, dst3], axis=2)
    x_pad = jnp.pad(x, ((0, NP - N), (0, 0)))

    degp = _deg_kernel(dst_flat)

    hp1 = _tc_mm1(degp, x_pad, W1)
    agg1 = _scatter_kernel(hp1, ei3)
    hp2 = _tc_mid(degp, agg1, b1.reshape(1, D), W2)
    agg2 = _scatter_kernel(hp2, ei3)
    out = _tc_head(degp, agg2, b2.reshape(1, D), Wf1, bf1.reshape(1, 128),
                   Wf2, bf2.reshape(1, 64))
    return out[:N]
